# baseline (speedup 1.0000x reference)
import functools

import numpy as np
import jax
import jax.numpy as jnp
from jax import lax
from jax.experimental import pallas as pl
from jax.experimental.pallas import tpu as pltpu
from jax.experimental.pallas import tpu_sc as plsc

_N = 768
_H = _W = 224
_D0 = _H * _W
_C = 2
_B = _N * _C
_D = _D0 // _C
_NC = 2
_NS = 16
_NW = _NC * _NS
_CPW = _B // _NW
_NBUF = 4
_STEPS = _CPW // _NBUF

_PERM = np.asarray(jax.random.permutation(jax.random.key(42), _N))
_IDX = (_PERM[:, None] * _C + np.arange(_C)[None, :]).reshape(_B).astype(np.int32)
_IDX8 = np.repeat(_IDX, 8)


def _swap_body(x_hbm, idx_hbm, out_hbm, idx_v, b0, b1, b2, b3, in_sems, out_sems):
    bufs = (b0, b1, b2, b3)
    wid = lax.axis_index("s") * _NC + lax.axis_index("c")
    base = wid * _CPW
    pltpu.sync_copy(idx_hbm.at[pl.ds(pl.multiple_of(base * 8, 8), _CPW * 8)], idx_v)

    def gather(k, b):
        return pltpu.make_async_copy(
            x_hbm.at[idx_v.at[pl.ds(pl.multiple_of(k * 8, 8), 1)]],
            bufs[b],
            in_sems.at[b],
        )

    def put(k, b):
        return pltpu.make_async_copy(
            bufs[b],
            out_hbm.at[pl.ds(base + k, 1)],
            out_sems.at[b],
        )

    for b in range(_NBUF):
        gather(b, b).start()

    def step(s, carry):
        k0 = s * _NBUF
        for b in range(_NBUF):
            k = k0 + b
            gather(k, b).wait()
            put(k, b).start()
            nk = k + _NBUF

            @pl.when(nk < _CPW)
            def _():
                put(k, b).wait()
                gather(nk, b).start()

        return carry

    lax.fori_loop(0, _STEPS, step, None)

    for b in range(_NBUF):
        put(_CPW - _NBUF + b, b).wait()


@functools.cache
def _swap():
    return pl.kernel(
        _swap_body,
        out_type=jax.ShapeDtypeStruct((_B, _D), jnp.float32),
        mesh=plsc.VectorSubcoreMesh(
            core_axis_name="c", subcore_axis_name="s",
            num_cores=_NC, num_subcores=_NS,
        ),
        scratch_types=[
            pltpu.VMEM((_CPW * 8,), jnp.int32),
            pltpu.VMEM((1, _D), jnp.float32),
            pltpu.VMEM((1, _D), jnp.float32),
            pltpu.VMEM((1, _D), jnp.float32),
            pltpu.VMEM((1, _D), jnp.float32),
            pltpu.SemaphoreType.DMA((_NBUF,)),
            pltpu.SemaphoreType.DMA((_NBUF,)),
        ],
    )


def kernel(x):
    xf = x.reshape(_B, _D)
    out = _swap()(xf, jnp.asarray(_IDX8))
    return out.reshape(_N, _H, _W)


# speedup vs baseline: 1.0003x; 1.0003x over previous
import functools

import numpy as np
import jax
import jax.numpy as jnp
from jax import lax
from jax.experimental import pallas as pl
from jax.experimental.pallas import tpu as pltpu
from jax.experimental.pallas import tpu_sc as plsc

_N = 768
_H = _W = 224
_D0 = _H * _W
_C = 2
_B = _N * _C
_D = _D0 // _C
_NC = 2
_NS = 16
_NW = _NC * _NS
_CPW = _B // _NW
_NBUF = 4
_LA = _NBUF // 2
_STEPS = _CPW // _NBUF

_PERM = np.asarray(jax.random.permutation(jax.random.key(42), _N))
_IDX = (_PERM[:, None] * _C + np.arange(_C)[None, :]).reshape(_B).astype(np.int32)
_IDX8 = np.repeat(_IDX, 8)


def _swap_body(x_hbm, idx_hbm, out_hbm, idx_v, b0, b1, b2, b3, in_sems, out_sems):
    bufs = (b0, b1, b2, b3)
    wid = lax.axis_index("s") * _NC + lax.axis_index("c")
    base = wid * _CPW
    pltpu.sync_copy(idx_hbm.at[pl.ds(pl.multiple_of(base * 8, 8), _CPW * 8)], idx_v)

    def gather(k, b):
        return pltpu.make_async_copy(
            x_hbm.at[idx_v.at[pl.ds(pl.multiple_of(k * 8, 8), 1)]],
            bufs[b],
            in_sems.at[b],
        )

    def put(k, b):
        return pltpu.make_async_copy(
            bufs[b],
            out_hbm.at[pl.ds(base + k, 1)],
            out_sems.at[b],
        )

    for j in range(_LA):
        gather(j, j % _NBUF).start()

    def emit(i, b, jb, first, last):
        j = i + _LA
        if not last:
            if not (first and b < _NBUF - _LA):
                put(j - _NBUF, jb).wait()
            gather(j, jb).start()
        elif b < _NBUF - _LA:
            put(j - _NBUF, jb).wait()
            gather(j, jb).start()
        gather(i, b).wait()
        put(i, b).start()

    for b in range(_NBUF):
        emit(b, b, (b + _LA) % _NBUF, True, False)

    def step(g, carry):
        for b in range(_NBUF):
            emit(g * _NBUF + b, b, (b + _LA) % _NBUF, False, False)
        return carry

    lax.fori_loop(1, _STEPS - 1, step, None)

    for b in range(_NBUF):
        emit((_STEPS - 1) * _NBUF + b, b, (b + _LA) % _NBUF, False, True)

    for b in range(_LA, _NBUF):
        put((_STEPS - 1) * _NBUF + b, b).wait()


@functools.cache
def _swap():
    return pl.kernel(
        _swap_body,
        out_type=jax.ShapeDtypeStruct((_B, _D), jnp.float32),
        mesh=plsc.VectorSubcoreMesh(
            core_axis_name="c", subcore_axis_name="s",
            num_cores=_NC, num_subcores=_NS,
        ),
        scratch_types=[
            pltpu.VMEM((_CPW * 8,), jnp.int32),
            pltpu.VMEM((1, _D), jnp.float32),
            pltpu.VMEM((1, _D), jnp.float32),
            pltpu.VMEM((1, _D), jnp.float32),
            pltpu.VMEM((1, _D), jnp.float32),
            pltpu.SemaphoreType.DMA((_NBUF,)),
            pltpu.SemaphoreType.DMA((_NBUF,)),
        ],
    )


def kernel(x):
    xf = x.reshape(_B, _D)
    out = _swap()(xf, jnp.asarray(_IDX8))
    return out.reshape(_N, _H, _W)
